# baseline (speedup 1.0000x reference)
import functools

import jax
import jax.numpy as jnp
from jax import lax
from jax.experimental import pallas as pl
from jax.experimental.pallas import tpu as pltpu
from jax.experimental.pallas import tpu_sc as plsc

B, C, T, S, K, OC = 8, 192, 1024, 256, 9, 192
OCP = 256
KP = 16
NW = 32
NH = 2
HB = B // NH
WPB = NW // HB
TW = T // WPB


def _dense_body(x_ref, xT_ref, yT_ref, wkT_ref, wv_ref, wq_ref, w2f_ref,
                biasp_ref, idxmap_ref, u_ref, idx9_ref):
    b = pl.program_id(0)
    f32, bf16 = jnp.float32, jnp.bfloat16

    def dot(a, bm):
        return jnp.dot(a.astype(bf16), bm.astype(bf16), preferred_element_type=f32)

    k_ = dot(x_ref[0], wkT_ref[...])
    vT = dot(wv_ref[...], xT_ref[0])
    qT = dot(wq_ref[...], yT_ref[0])
    u_ref[0] = dot(vT, w2f_ref[...]) + biasp_ref[...]
    kn_ = k_ / jnp.maximum(jnp.sqrt(jnp.sum(k_ * k_, axis=0, keepdims=True)), 1e-12)
    qnT = qT / jnp.maximum(jnp.sqrt(jnp.sum(qT * qT, axis=1, keepdims=True)), 1e-12)
    simT = jnp.maximum(dot(qnT, kn_), 0.0)

    iota_s = lax.broadcasted_iota(jnp.int32, (S, T), 0)
    iota_t = lax.broadcasted_iota(jnp.int32, (1, T), 1)
    idxmap = jnp.broadcast_to(idxmap_ref[...], (S, T))
    base = b * T
    idx9_ref[0, pl.ds(0, 1), :] = (base + iota_t) * K
    for j in range(K - 1):
        m = jnp.max(simT, axis=0, keepdims=True)
        cand = jnp.where(simT >= m, iota_s, S)
        amin = jnp.min(cand, axis=0, keepdims=True)
        sel = iota_s == amin
        mapped = jnp.sum(jnp.where(sel, idxmap, 0), axis=0, keepdims=True)
        simT = jnp.where(sel, -1.0, simT)
        idx9_ref[0, pl.ds(j + 1, 1), :] = (base + mapped) * K + (j + 1)


CO = 32
NCH = TW // CO
NV = OC // 16


def _sc_gather_body(u_hbm, idx_hbm, out_hbm, idx_v, rows_v, out_v, sem0):
    wid = lax.axis_index("s") * 2 + lax.axis_index("c")
    b = wid // WPB
    t0 = (wid % WPB) * TW
    pltpu.sync_copy(idx_hbm.at[pl.ds(b * KP, KP), pl.ds(t0, TW)], idx_v)
    for c in range(NCH):
        cps = [pltpu.async_copy(u_hbm.at[idx_v.at[k, pl.ds(c * CO, CO)]],
                                rows_v.at[pl.ds(k * CO, CO), :], sem0)
               for k in range(K)]
        for cp in cps:
            cp.wait()

        def body(i):
            for j in range(NV):
                acc = rows_v[i, pl.ds(j * 16, 16)]
                for k in range(1, K):
                    acc = acc + rows_v[k * CO + i, pl.ds(j * 16, 16)]
                out_v[i, pl.ds(j * 16, 16)] = acc

        lax.fori_loop(0, CO, lambda i, _: (body(i), 0)[1], 0)
        pltpu.sync_copy(out_v, out_hbm.at[pl.ds(wid * TW + c * CO, CO), :])


def _sc_call(u_flat, idx2d):
    mesh = plsc.VectorSubcoreMesh(core_axis_name="c", subcore_axis_name="s")
    sc_fn = functools.partial(
        pl.kernel,
        out_type=jax.ShapeDtypeStruct((HB * T, OCP), jnp.float32),
        mesh=mesh,
        scratch_types=[
            pltpu.VMEM((KP, TW), jnp.int32),
            pltpu.VMEM((K * CO, OCP), jnp.float32),
            pltpu.VMEM((CO, OCP), jnp.float32),
            pltpu.SemaphoreType.DMA,
        ],
    )(_sc_gather_body)
    return sc_fn(u_flat, idx2d)


def kernel(x, y, indices, Wq, Wk, Wv, conv_w, conv_b):
    xT = jnp.transpose(x, (0, 2, 1))
    yT = jnp.transpose(y, (0, 2, 1))
    wkT = jnp.transpose(Wk)
    w2f = jnp.transpose(conv_w, (1, 2, 0))
    w2f = jnp.pad(w2f, ((0, 0), (0, 0), (0, OCP - OC))).reshape(C, K * OCP)
    biasp = jnp.concatenate(
        [conv_b, jnp.zeros(K * OCP - OC, jnp.float32)]).reshape(1, K * OCP)
    idxmap = indices.astype(jnp.int32).reshape(S, 1)

    halves = []
    for h in range(NH):
        sl = slice(h * HB, (h + 1) * HB)
        u, idx9 = pl.pallas_call(
            _dense_body,
            grid=(HB,),
            in_specs=[
                pl.BlockSpec((1, C, T), lambda b: (b, 0, 0)),
                pl.BlockSpec((1, T, C), lambda b: (b, 0, 0)),
                pl.BlockSpec((1, S, C), lambda b: (b, 0, 0)),
                pl.BlockSpec((T, T), lambda b: (0, 0)),
                pl.BlockSpec((T, T), lambda b: (0, 0)),
                pl.BlockSpec((S, S), lambda b: (0, 0)),
                pl.BlockSpec((C, K * OCP), lambda b: (0, 0)),
                pl.BlockSpec((1, K * OCP), lambda b: (0, 0)),
                pl.BlockSpec((S, 1), lambda b: (0, 0)),
            ],
            out_specs=[
                pl.BlockSpec((1, T, K * OCP), lambda b: (b, 0, 0)),
                pl.BlockSpec((1, KP, T), lambda b: (b, 0, 0)),
            ],
            out_shape=[
                jax.ShapeDtypeStruct((HB, T, K * OCP), jnp.float32),
                jax.ShapeDtypeStruct((HB, KP, T), jnp.int32),
            ],
        )(x[sl], xT[sl], yT[sl], wkT, Wv, Wq, w2f, biasp, idxmap)
        halves.append((u.reshape(HB * T * K, OCP), idx9.reshape(HB * KP, T)))

    out_rows = jnp.concatenate([_sc_call(u_flat, idx2d)
                                for u_flat, idx2d in halves], axis=0)

    return jnp.transpose(out_rows.reshape(B, T, OCP)[:, :, :OC], (0, 2, 1))


# speedup vs baseline: 1.0257x; 1.0257x over previous
import functools

import jax
import jax.numpy as jnp
from jax import lax
from jax.experimental import pallas as pl
from jax.experimental.pallas import tpu as pltpu
from jax.experimental.pallas import tpu_sc as plsc

B, C, T, S, K, OC = 8, 192, 1024, 256, 9, 192
OCP = 256
KP = 16
NW = 32
NH = 1
HB = B // NH
WPB = NW // HB
TW = T // WPB


def _dense_body(x_ref, xT_ref, yT_ref, wkT_ref, wv_ref, wq_ref, w2f_ref,
                biasp_ref, idxmap_ref, u_ref, idx9_ref):
    b = pl.program_id(0)
    f32, bf16 = jnp.float32, jnp.bfloat16

    def dot(a, bm):
        return jnp.dot(a.astype(bf16), bm.astype(bf16), preferred_element_type=f32)

    k_ = dot(x_ref[0], wkT_ref[...])
    vT = dot(wv_ref[...], xT_ref[0])
    qT = dot(wq_ref[...], yT_ref[0])
    u_ref[0] = dot(vT, w2f_ref[...]) + biasp_ref[...]
    kn_ = k_ / jnp.maximum(jnp.sqrt(jnp.sum(k_ * k_, axis=0, keepdims=True)), 1e-12)
    qnT = qT / jnp.maximum(jnp.sqrt(jnp.sum(qT * qT, axis=1, keepdims=True)), 1e-12)
    simT = jnp.maximum(dot(qnT, kn_), 0.0)

    iota_s = lax.broadcasted_iota(jnp.int32, (S, T), 0)
    iota_t = lax.broadcasted_iota(jnp.int32, (1, T), 1)
    idxmap = jnp.broadcast_to(idxmap_ref[...], (S, T))
    base = b * T
    idx9_ref[0, pl.ds(0, 1), :] = (base + iota_t) * K
    for j in range(K - 1):
        m = jnp.max(simT, axis=0, keepdims=True)
        cand = jnp.where(simT >= m, iota_s, S)
        amin = jnp.min(cand, axis=0, keepdims=True)
        sel = iota_s == amin
        mapped = jnp.sum(jnp.where(sel, idxmap, 0), axis=0, keepdims=True)
        simT = jnp.where(sel, -1.0, simT)
        idx9_ref[0, pl.ds(j + 1, 1), :] = (base + mapped) * K + (j + 1)


CO = 32
NCH = TW // CO
NV = OC // 16


def _sc_gather_body(u_hbm, idx_hbm, out_hbm, idx_v, rows_v, out_v, sem0):
    wid = lax.axis_index("s") * 2 + lax.axis_index("c")
    b = wid // WPB
    t0 = (wid % WPB) * TW
    pltpu.sync_copy(idx_hbm.at[pl.ds(b * KP, KP), pl.ds(t0, TW)], idx_v)
    for c in range(NCH):
        cps = [pltpu.async_copy(u_hbm.at[idx_v.at[k, pl.ds(c * CO, CO)]],
                                rows_v.at[pl.ds(k * CO, CO), :], sem0)
               for k in range(K)]
        for cp in cps:
            cp.wait()

        def body(i):
            for j in range(NV):
                acc = rows_v[i, pl.ds(j * 16, 16)]
                for k in range(1, K):
                    acc = acc + rows_v[k * CO + i, pl.ds(j * 16, 16)]
                out_v[i, pl.ds(j * 16, 16)] = acc

        lax.fori_loop(0, CO, lambda i, _: (body(i), 0)[1], 0)
        pltpu.sync_copy(out_v, out_hbm.at[pl.ds(wid * TW + c * CO, CO), :])


def _sc_call(u_flat, idx2d):
    mesh = plsc.VectorSubcoreMesh(core_axis_name="c", subcore_axis_name="s")
    sc_fn = functools.partial(
        pl.kernel,
        out_type=jax.ShapeDtypeStruct((HB * T, OCP), jnp.float32),
        mesh=mesh,
        scratch_types=[
            pltpu.VMEM((KP, TW), jnp.int32),
            pltpu.VMEM((K * CO, OCP), jnp.float32),
            pltpu.VMEM((CO, OCP), jnp.float32),
            pltpu.SemaphoreType.DMA,
        ],
    )(_sc_gather_body)
    return sc_fn(u_flat, idx2d)


def kernel(x, y, indices, Wq, Wk, Wv, conv_w, conv_b):
    xT = jnp.transpose(x, (0, 2, 1))
    yT = jnp.transpose(y, (0, 2, 1))
    wkT = jnp.transpose(Wk)
    w2f = jnp.transpose(conv_w, (1, 2, 0))
    w2f = jnp.pad(w2f, ((0, 0), (0, 0), (0, OCP - OC))).reshape(C, K * OCP)
    biasp = jnp.concatenate(
        [conv_b, jnp.zeros(K * OCP - OC, jnp.float32)]).reshape(1, K * OCP)
    idxmap = indices.astype(jnp.int32).reshape(S, 1)

    halves = []
    for h in range(NH):
        sl = slice(h * HB, (h + 1) * HB)
        u, idx9 = pl.pallas_call(
            _dense_body,
            grid=(HB,),
            in_specs=[
                pl.BlockSpec((1, C, T), lambda b: (b, 0, 0)),
                pl.BlockSpec((1, T, C), lambda b: (b, 0, 0)),
                pl.BlockSpec((1, S, C), lambda b: (b, 0, 0)),
                pl.BlockSpec((T, T), lambda b: (0, 0)),
                pl.BlockSpec((T, T), lambda b: (0, 0)),
                pl.BlockSpec((S, S), lambda b: (0, 0)),
                pl.BlockSpec((C, K * OCP), lambda b: (0, 0)),
                pl.BlockSpec((1, K * OCP), lambda b: (0, 0)),
                pl.BlockSpec((S, 1), lambda b: (0, 0)),
            ],
            out_specs=[
                pl.BlockSpec((1, T, K * OCP), lambda b: (b, 0, 0)),
                pl.BlockSpec((1, KP, T), lambda b: (b, 0, 0)),
            ],
            out_shape=[
                jax.ShapeDtypeStruct((HB, T, K * OCP), jnp.float32),
                jax.ShapeDtypeStruct((HB, KP, T), jnp.int32),
            ],
        )(x[sl], xT[sl], yT[sl], wkT, Wv, Wq, w2f, biasp, idxmap)
        halves.append((u.reshape(HB * T * K, OCP), idx9.reshape(HB * KP, T)))

    out_rows = jnp.concatenate([_sc_call(u_flat, idx2d)
                                for u_flat, idx2d in halves], axis=0)

    return jnp.transpose(out_rows.reshape(B, T, OCP)[:, :, :OC], (0, 2, 1))


# speedup vs baseline: 1.0338x; 1.0079x over previous
import functools

import jax
import jax.numpy as jnp
from jax import lax
from jax.experimental import pallas as pl
from jax.experimental.pallas import tpu as pltpu
from jax.experimental.pallas import tpu_sc as plsc

B, C, T, S, K, OC = 8, 192, 1024, 256, 9, 192
OCP = 256
KP = 16
NW = 32
NH = 1
HB = B // NH
WPB = NW // HB
TW = T // WPB


def _dense_body(x_ref, y_ref, wkT_ref, wvT_ref, wqT_ref, w2f_ref,
                biasp_ref, idxmap_ref, u_ref, idx9_ref):
    b = pl.program_id(0)
    f32, bf16 = jnp.float32, jnp.bfloat16

    def dot(a, bm):
        return jnp.dot(a.astype(bf16), bm.astype(bf16), preferred_element_type=f32)

    def dot0(a, bm):
        return lax.dot_general(a.astype(bf16), bm.astype(bf16),
                               (((0,), (0,)), ((), ())),
                               preferred_element_type=f32)

    k_ = dot(x_ref[0], wkT_ref[...])
    v_ = dot(x_ref[0], wvT_ref[...])
    q_ = dot(y_ref[0], wqT_ref[...])
    u_ref[0] = dot0(v_, w2f_ref[...]) + biasp_ref[...]
    kn_ = k_ / jnp.maximum(jnp.sqrt(jnp.sum(k_ * k_, axis=0, keepdims=True)), 1e-12)
    qn_ = q_ / jnp.maximum(jnp.sqrt(jnp.sum(q_ * q_, axis=0, keepdims=True)), 1e-12)
    simT = jnp.maximum(dot0(qn_, kn_), 0.0)

    iota_s = lax.broadcasted_iota(jnp.int32, (S, T), 0)
    iota_t = lax.broadcasted_iota(jnp.int32, (1, T), 1)
    idxmap = jnp.broadcast_to(idxmap_ref[...], (S, T))
    base = b * T
    idx9_ref[0, pl.ds(0, 1), :] = (base + iota_t) * K
    for j in range(K - 1):
        m = jnp.max(simT, axis=0, keepdims=True)
        cand = jnp.where(simT >= m, iota_s, S)
        amin = jnp.min(cand, axis=0, keepdims=True)
        sel = iota_s == amin
        mapped = jnp.sum(jnp.where(sel, idxmap, 0), axis=0, keepdims=True)
        simT = jnp.where(sel, -1.0, simT)
        idx9_ref[0, pl.ds(j + 1, 1), :] = (base + mapped) * K + (j + 1)


CO = 32
NCH = TW // CO
NV = OC // 16


def _sc_gather_body(u_hbm, idx_hbm, out_hbm, idx_v, rows_v, out_v, sem0):
    wid = lax.axis_index("s") * 2 + lax.axis_index("c")
    b = wid // WPB
    t0 = (wid % WPB) * TW
    pltpu.sync_copy(idx_hbm.at[pl.ds(b * KP, KP), pl.ds(t0, TW)], idx_v)
    for c in range(NCH):
        cps = [pltpu.async_copy(u_hbm.at[idx_v.at[k, pl.ds(c * CO, CO)]],
                                rows_v.at[pl.ds(k * CO, CO), :], sem0)
               for k in range(K)]
        for cp in cps:
            cp.wait()

        def body(i):
            for j in range(NV):
                acc = rows_v[i, pl.ds(j * 16, 16)]
                for k in range(1, K):
                    acc = acc + rows_v[k * CO + i, pl.ds(j * 16, 16)]
                out_v[i, pl.ds(j * 16, 16)] = acc

        lax.fori_loop(0, CO, lambda i, _: (body(i), 0)[1], 0)
        pltpu.sync_copy(out_v, out_hbm.at[pl.ds(wid * TW + c * CO, CO), :])


def _sc_call(u_flat, idx2d):
    mesh = plsc.VectorSubcoreMesh(core_axis_name="c", subcore_axis_name="s")
    sc_fn = functools.partial(
        pl.kernel,
        out_type=jax.ShapeDtypeStruct((HB * T, OCP), jnp.float32),
        mesh=mesh,
        scratch_types=[
            pltpu.VMEM((KP, TW), jnp.int32),
            pltpu.VMEM((K * CO, OCP), jnp.float32),
            pltpu.VMEM((CO, OCP), jnp.float32),
            pltpu.SemaphoreType.DMA,
        ],
    )(_sc_gather_body)
    return sc_fn(u_flat, idx2d)


def kernel(x, y, indices, Wq, Wk, Wv, conv_w, conv_b):
    wkT = jnp.transpose(Wk)
    wvT = jnp.transpose(Wv)
    wqT = jnp.transpose(Wq)
    w2f = jnp.transpose(conv_w, (1, 2, 0))
    w2f = jnp.pad(w2f, ((0, 0), (0, 0), (0, OCP - OC))).reshape(C, K * OCP)
    biasp = jnp.concatenate(
        [conv_b, jnp.zeros(K * OCP - OC, jnp.float32)]).reshape(1, K * OCP)
    idxmap = indices.astype(jnp.int32).reshape(S, 1)

    halves = []
    for h in range(NH):
        sl = slice(h * HB, (h + 1) * HB)
        u, idx9 = pl.pallas_call(
            _dense_body,
            grid=(HB,),
            in_specs=[
                pl.BlockSpec((1, C, T), lambda b: (b, 0, 0)),
                pl.BlockSpec((1, C, S), lambda b: (b, 0, 0)),
                pl.BlockSpec((T, T), lambda b: (0, 0)),
                pl.BlockSpec((T, T), lambda b: (0, 0)),
                pl.BlockSpec((S, S), lambda b: (0, 0)),
                pl.BlockSpec((C, K * OCP), lambda b: (0, 0)),
                pl.BlockSpec((1, K * OCP), lambda b: (0, 0)),
                pl.BlockSpec((S, 1), lambda b: (0, 0)),
            ],
            out_specs=[
                pl.BlockSpec((1, T, K * OCP), lambda b: (b, 0, 0)),
                pl.BlockSpec((1, KP, T), lambda b: (b, 0, 0)),
            ],
            out_shape=[
                jax.ShapeDtypeStruct((HB, T, K * OCP), jnp.float32),
                jax.ShapeDtypeStruct((HB, KP, T), jnp.int32),
            ],
        )(x[sl], y[sl], wkT, wvT, wqT, w2f, biasp, idxmap)
        halves.append((u.reshape(HB * T * K, OCP), idx9.reshape(HB * KP, T)))

    out_rows = jnp.concatenate([_sc_call(u_flat, idx2d)
                                for u_flat, idx2d in halves], axis=0)

    return jnp.transpose(out_rows.reshape(B, T, OCP)[:, :, :OC], (0, 2, 1))


# speedup vs baseline: 1.0960x; 1.0602x over previous
import functools

import jax
import jax.numpy as jnp
from jax import lax
from jax.experimental import pallas as pl
from jax.experimental.pallas import tpu as pltpu
from jax.experimental.pallas import tpu_sc as plsc

B, C, T, S, K, OC = 8, 192, 1024, 256, 9, 192
OCP = 256
KP = 16
NW = 32
NH = 1
HB = B // NH
WPB = NW // HB
TW = T // WPB


def _dense_body(x_ref, y_ref, wkT_ref, wvT_ref, wqT_ref, w2f_ref,
                biasp_ref, idxmap_ref, u_ref, idx9_ref):
    b = pl.program_id(0)
    f32, bf16 = jnp.float32, jnp.bfloat16

    def dot(a, bm):
        return jnp.dot(a.astype(bf16), bm.astype(bf16), preferred_element_type=f32)

    def dot0(a, bm):
        return lax.dot_general(a.astype(bf16), bm.astype(bf16),
                               (((0,), (0,)), ((), ())),
                               preferred_element_type=f32)

    k_ = dot(x_ref[0], wkT_ref[...])
    v_ = dot(x_ref[0], wvT_ref[...])
    q_ = dot(y_ref[0], wqT_ref[...])
    u_ref[0] = dot0(v_, w2f_ref[...]) + biasp_ref[...]
    kn_ = k_ / jnp.maximum(jnp.sqrt(jnp.sum(k_ * k_, axis=0, keepdims=True)), 1e-12)
    qn_ = q_ / jnp.maximum(jnp.sqrt(jnp.sum(q_ * q_, axis=0, keepdims=True)), 1e-12)
    simT = jnp.maximum(dot0(qn_, kn_), 0.0)

    iota_s = lax.broadcasted_iota(jnp.int32, (S, T), 0)
    iota_t = lax.broadcasted_iota(jnp.int32, (1, T), 1)
    idxmap = jnp.broadcast_to(idxmap_ref[...], (S, T))
    base = b * T
    idx9_ref[0, pl.ds(0, 1), :] = (base + iota_t) * K
    for j in range(K - 1):
        m = jnp.max(simT, axis=0, keepdims=True)
        cand = jnp.where(simT >= m, iota_s, S)
        amin = jnp.min(cand, axis=0, keepdims=True)
        sel = iota_s == amin
        mapped = jnp.sum(jnp.where(sel, idxmap, 0), axis=0, keepdims=True)
        simT = jnp.where(sel, -1.0, simT)
        idx9_ref[0, pl.ds(j + 1, 1), :] = (base + mapped) * K + (j + 1)


CO = 16
NCH = TW // CO
NV = OC // 16


def _sc_gather_body(u_hbm, idx_hbm, out_hbm, idx_v, rows_v, out_v, sem0, sem1):
    wid = lax.axis_index("s") * 2 + lax.axis_index("c")
    b = wid // WPB
    t0 = (wid % WPB) * TW
    pltpu.sync_copy(idx_hbm.at[pl.ds(b * KP, KP), pl.ds(t0, TW)], idx_v)
    sems = (sem0, sem1)

    def fire(c):
        off = (c % 2) * K * CO
        return [pltpu.async_copy(u_hbm.at[idx_v.at[k, pl.ds(c * CO, CO)]],
                                 rows_v.at[pl.ds(off + k * CO, CO), :], sems[c % 2])
                for k in range(K)]

    pend = fire(0)
    for c in range(NCH):
        for cp in pend:
            cp.wait()
        if c + 1 < NCH:
            pend = fire(c + 1)
        off = (c % 2) * K * CO

        def body(i):
            for j in range(NV):
                acc = rows_v[off + i, pl.ds(j * 16, 16)]
                for k in range(1, K):
                    acc = acc + rows_v[off + k * CO + i, pl.ds(j * 16, 16)]
                out_v[i, pl.ds(j * 16, 16)] = acc

        lax.fori_loop(0, CO, lambda i, _: (body(i), 0)[1], 0)
        pltpu.sync_copy(out_v, out_hbm.at[pl.ds(wid * TW + c * CO, CO), :])


def _sc_call(u_flat, idx2d):
    mesh = plsc.VectorSubcoreMesh(core_axis_name="c", subcore_axis_name="s")
    sc_fn = functools.partial(
        pl.kernel,
        out_type=jax.ShapeDtypeStruct((HB * T, OCP), jnp.float32),
        mesh=mesh,
        scratch_types=[
            pltpu.VMEM((KP, TW), jnp.int32),
            pltpu.VMEM((2 * K * CO, OCP), jnp.float32),
            pltpu.VMEM((CO, OCP), jnp.float32),
            pltpu.SemaphoreType.DMA,
            pltpu.SemaphoreType.DMA,
        ],
    )(_sc_gather_body)
    return sc_fn(u_flat, idx2d)


def kernel(x, y, indices, Wq, Wk, Wv, conv_w, conv_b):
    wkT = jnp.transpose(Wk)
    wvT = jnp.transpose(Wv)
    wqT = jnp.transpose(Wq)
    w2f = jnp.transpose(conv_w, (1, 2, 0))
    w2f = jnp.pad(w2f, ((0, 0), (0, 0), (0, OCP - OC))).reshape(C, K * OCP)
    biasp = jnp.concatenate(
        [conv_b, jnp.zeros(K * OCP - OC, jnp.float32)]).reshape(1, K * OCP)
    idxmap = indices.astype(jnp.int32).reshape(S, 1)

    halves = []
    for h in range(NH):
        sl = slice(h * HB, (h + 1) * HB)
        u, idx9 = pl.pallas_call(
            _dense_body,
            grid=(HB,),
            in_specs=[
                pl.BlockSpec((1, C, T), lambda b: (b, 0, 0)),
                pl.BlockSpec((1, C, S), lambda b: (b, 0, 0)),
                pl.BlockSpec((T, T), lambda b: (0, 0)),
                pl.BlockSpec((T, T), lambda b: (0, 0)),
                pl.BlockSpec((S, S), lambda b: (0, 0)),
                pl.BlockSpec((C, K * OCP), lambda b: (0, 0)),
                pl.BlockSpec((1, K * OCP), lambda b: (0, 0)),
                pl.BlockSpec((S, 1), lambda b: (0, 0)),
            ],
            out_specs=[
                pl.BlockSpec((1, T, K * OCP), lambda b: (b, 0, 0)),
                pl.BlockSpec((1, KP, T), lambda b: (b, 0, 0)),
            ],
            out_shape=[
                jax.ShapeDtypeStruct((HB, T, K * OCP), jnp.float32),
                jax.ShapeDtypeStruct((HB, KP, T), jnp.int32),
            ],
        )(x[sl], y[sl], wkT, wvT, wqT, w2f, biasp, idxmap)
        halves.append((u.reshape(HB * T * K, OCP), idx9.reshape(HB * KP, T)))

    out_rows = jnp.concatenate([_sc_call(u_flat, idx2d)
                                for u_flat, idx2d in halves], axis=0)

    return jnp.transpose(out_rows.reshape(B, T, OCP)[:, :, :OC], (0, 2, 1))
